# baseline (speedup 1.0000x reference)
import functools

import jax
import jax.numpy as jnp
from jax import lax
from jax.experimental import pallas as pl
from jax.experimental.pallas import tpu as pltpu
from jax.experimental.pallas import tpu_sc as plsc

NC = 2
NS = 16
NW = NC * NS
C = 128
G = 8


def _pre_sc(E, EP, R, NSEGP):
    ROWS = EP // C
    RPT_A = ROWS // NS
    RPT_B = ROWS // NW
    ZB = 1024
    PS = NSEGP // NS

    def body(srcp, dstp, etp, vald, s_out, g_out,
             counts_sh, srcb, dbuf, ebuf, vbuf, segb, gidxb, cntb, soutb,
             zb, sem):
        cid = lax.axis_index("c")
        sid = lax.axis_index("s")
        w = sid * NC + cid

        for k in range(ZB // 16):
            zb[pl.ds(k * 16, 16)] = jnp.zeros((16,), jnp.float32)
        def body_z(z, c):
            pltpu.sync_copy(zb, counts_sh.at[pl.ds(sid * PS + z * ZB, ZB)])
            return c
        lax.fori_loop(0, PS // ZB, body_z, 0)
        plsc.subcore_barrier()

        def body_a(g, c):
            row0 = sid * RPT_A + g * G
            pltpu.sync_copy(dstp.at[pl.ds(row0, G)], dbuf)
            pltpu.sync_copy(etp.at[pl.ds(row0, G)], ebuf)
            pltpu.sync_copy(vald.at[pl.ds(row0, G)], vbuf)
            for j in range(G):
                for k in range(C // 16):
                    sl = pl.ds(k * 16, 16)
                    segb[j, sl] = dbuf[j, sl] * R + ebuf[j, sl]
            cps = [pltpu.async_copy(vbuf.at[j], counts_sh.at[segb.at[j]],
                                    sem, add=True) for j in range(G)]
            for cp in cps:
                cp.wait()
            return c
        lax.fori_loop(0, RPT_A // G, body_a, 0)
        plsc.subcore_barrier()

        def body_b(g, c):
            row0 = w * RPT_B + g * G
            pltpu.sync_copy(srcp.at[pl.ds(row0, G)], srcb)
            pltpu.sync_copy(dstp.at[pl.ds(row0, G)], dbuf)
            pltpu.sync_copy(etp.at[pl.ds(row0, G)], ebuf)
            pltpu.sync_copy(vald.at[pl.ds(row0, G)], vbuf)
            for j in range(G):
                for k in range(C // 16):
                    sl = pl.ds(k * 16, 16)
                    segb[j, sl] = dbuf[j, sl] * R + ebuf[j, sl]
                    gidxb[j, sl] = srcb[j, sl] * R + ebuf[j, sl]
            cps = [pltpu.async_copy(counts_sh.at[segb.at[j]], cntb.at[j], sem)
                   for j in range(G)]
            for cp in cps:
                cp.wait()
            for j in range(G):
                for k in range(C // 16):
                    sl = pl.ds(k * 16, 16)
                    soutb[j, sl] = vbuf[j, sl] / jnp.maximum(cntb[j, sl], 1.0)
            pltpu.sync_copy(soutb, s_out.at[pl.ds(row0, G)])
            pltpu.sync_copy(gidxb, g_out.at[pl.ds(row0, G)])
            return c
        lax.fori_loop(0, RPT_B // G, body_b, 0)

    mesh = plsc.VectorSubcoreMesh(core_axis_name="c", subcore_axis_name="s")
    return pl.kernel(
        body,
        out_type=(jax.ShapeDtypeStruct((ROWS, C), jnp.float32),
                  jax.ShapeDtypeStruct((ROWS, C), jnp.int32)),
        mesh=mesh,
        compiler_params=pltpu.CompilerParams(use_tc_tiling_on_sc=False),
        scratch_types=[
            pltpu.VMEM_SHARED((NSEGP,), jnp.float32),
            pltpu.VMEM((G, C), jnp.int32),
            pltpu.VMEM((G, C), jnp.int32),
            pltpu.VMEM((G, C), jnp.int32),
            pltpu.VMEM((G, C), jnp.float32),
            pltpu.VMEM((G, C), jnp.int32),
            pltpu.VMEM((G, C), jnp.int32),
            pltpu.VMEM((G, C), jnp.float32),
            pltpu.VMEM((G, C), jnp.float32),
            pltpu.VMEM((ZB,), jnp.float32),
            pltpu.SemaphoreType.DMA,
        ],
    )


def _layer_sc(NP, D, EP, NR):
    ROWS = EP // C
    RPT = ROWS // NW
    NPT = NP // NS
    ZR = C

    def body(y, g2d, dstp, s2d, out,
             acc_sh, gbuf, dbuf, sbuf, gr0, gr1, fr0, fr1,
             sem0, sem1, sem_s):
        cid = lax.axis_index("c")
        sid = lax.axis_index("s")
        w = sid * NC + cid

        def body_z0(e, c):
            fr0[e, pl.ds(0, 16)] = jnp.zeros((16,), jnp.float32)
            fr0[e, pl.ds(16, 16)] = jnp.zeros((16,), jnp.float32)
            return c
        lax.fori_loop(0, ZR, body_z0, 0)
        def body_z(z, c):
            pltpu.sync_copy(fr0.at[pl.ds(0, ZR)],
                            acc_sh.at[pl.ds(sid * NPT + z * ZR, ZR)])
            return c
        lax.fori_loop(0, NPT // ZR, body_z, 0)
        plsc.subcore_barrier()

        gbufs = (gr0, gr1)
        fbufs = (fr0, fr1)
        sems = (sem0, sem1)
        himask = jnp.full((16,), -65536, jnp.int32)

        def scale(grows, frows, sbuf, j):
            def scale_body(k, c2):
                sv = sbuf[j, pl.ds(k * 16, 16)]
                for i in range(16):
                    e = k * 16 + i
                    si = sv[i]
                    wv = grows[e, pl.ds(0, 16)]
                    a = plsc.bitcast(wv << 16, jnp.float32)
                    bv = plsc.bitcast(wv & himask, jnp.float32)
                    frows[e, pl.ds(0, 16)] = a * si
                    frows[e, pl.ds(16, 16)] = bv * si
                return c2
            lax.fori_loop(0, C // 16, scale_body, 0)

        def body_g(g, c):
            row0 = w * RPT + g * G
            pltpu.sync_copy(g2d.at[pl.ds(row0, G)], gbuf)
            pltpu.sync_copy(dstp.at[pl.ds(row0, G)], dbuf)
            pltpu.sync_copy(s2d.at[pl.ds(row0, G)], sbuf)
            cp = pltpu.async_copy(y.at[gbuf.at[0]], gbufs[0], sems[0])
            scs = []
            for j in range(G):
                if j + 1 < G:
                    cp_n = pltpu.async_copy(y.at[gbuf.at[j + 1]],
                                            gbufs[(j + 1) % 2],
                                            sems[(j + 1) % 2])
                cp.wait()
                if j >= 2:
                    scs[j - 2].wait()
                scale(gbufs[j % 2], fbufs[j % 2], sbuf, j)
                scs.append(pltpu.async_copy(fbufs[j % 2],
                                            acc_sh.at[dbuf.at[j]],
                                            sem_s, add=True))
                if j + 1 < G:
                    cp = cp_n
            scs[G - 2].wait()
            scs[G - 1].wait()
            return c
        lax.fori_loop(0, RPT // G, body_g, 0)
        plsc.subcore_barrier()
        def body_o(z, c):
            off = sid * NPT + z * ZR
            pltpu.sync_copy(acc_sh.at[pl.ds(off, ZR)], fr0.at[pl.ds(0, ZR)])
            pltpu.sync_copy(fr0.at[pl.ds(0, ZR)], out.at[cid, pl.ds(off, ZR)])
            return c
        lax.fori_loop(0, NPT // ZR, body_o, 0)

    mesh = plsc.VectorSubcoreMesh(core_axis_name="c", subcore_axis_name="s")
    return pl.kernel(
        body,
        out_type=jax.ShapeDtypeStruct((NC, NP, D), jnp.float32),
        mesh=mesh,
        compiler_params=pltpu.CompilerParams(use_tc_tiling_on_sc=False,
                                             needs_layout_passes=False),
        scratch_types=[
            pltpu.VMEM_SHARED((NP, D), jnp.float32),
            pltpu.VMEM((G, C), jnp.int32),
            pltpu.VMEM((G, C), jnp.int32),
            pltpu.VMEM((G, C), jnp.float32),
            pltpu.VMEM((C, D // 2), jnp.int32),
            pltpu.VMEM((C, D // 2), jnp.int32),
            pltpu.VMEM((C, D), jnp.float32),
            pltpu.VMEM((C, D), jnp.float32),
            pltpu.SemaphoreType.DMA,
            pltpu.SemaphoreType.DMA,
            pltpu.SemaphoreType.DMA,
        ],
    )


def _mm_body(x_ref, w_ref, o_ref):
    o_ref[...] = lax.dot_general(
        x_ref[...], w_ref[...], (((1,), (0,)), ((), ())),
        precision=lax.Precision.HIGHEST,
        preferred_element_type=jnp.float32).astype(jnp.bfloat16)


def _matmul(x, w, bn):
    n, d = x.shape
    _, m = w.shape
    return pl.pallas_call(
        _mm_body,
        grid=(n // bn,),
        in_specs=[pl.BlockSpec((bn, d), lambda i: (i, 0)),
                  pl.BlockSpec((d, m), lambda i: (0, 0))],
        out_specs=pl.BlockSpec((bn, m), lambda i: (i, 0)),
        out_shape=jax.ShapeDtypeStruct((n, m), jnp.bfloat16),
    )(x, w)


def _comb_body(a_ref, x_ref, wr_ref, b_ref, o_ref, *, relu):
    v = (a_ref[0] + a_ref[1] + b_ref[...]
         + lax.dot_general(x_ref[...], wr_ref[...], (((1,), (0,)), ((), ())),
                           precision=lax.Precision.HIGHEST,
                           preferred_element_type=jnp.float32))
    o_ref[...] = jnp.maximum(v, 0.0) if relu else v


def _combine(part, x, wr, bvec, relu, bn):
    n, d = x.shape
    return pl.pallas_call(
        functools.partial(_comb_body, relu=relu),
        grid=(n // bn,),
        in_specs=[pl.BlockSpec((2, bn, d), lambda i: (0, i, 0)),
                  pl.BlockSpec((bn, d), lambda i: (i, 0)),
                  pl.BlockSpec((d, d), lambda i: (0, 0)),
                  pl.BlockSpec((1, d), lambda i: (0, 0))],
        out_specs=pl.BlockSpec((bn, d), lambda i: (i, 0)),
        out_shape=jax.ShapeDtypeStruct((n, d), jnp.float32),
    )(part, x, wr, bvec.reshape(1, d))


def kernel(edge_index, edge_type, node_emb, W, W_root, b):
    N, D = node_emb.shape
    L, R = W.shape[0], W.shape[1]
    E = edge_type.shape[0]
    NR = N * R

    EP = -(-E // (NW * G * C)) * (NW * G * C)
    ROWS = EP // C
    NSEGP = -(-NR // (NS * 1024)) * (NS * 1024)
    pad = EP - E

    src = edge_index[0]
    dst = edge_index[1]
    zi = jnp.zeros((pad,), jnp.int32)
    srcp = jnp.concatenate([src, zi]).reshape(ROWS, C)
    dstp = jnp.concatenate([dst, zi]).reshape(ROWS, C)
    etp = jnp.concatenate([edge_type, zi]).reshape(ROWS, C)
    vald = jnp.concatenate([jnp.ones((E,), jnp.float32),
                            jnp.zeros((pad,), jnp.float32)]).reshape(ROWS, C)

    s2d, g2d = _pre_sc(E, EP, R, NSEGP)(srcp, dstp, etp, vald)
    NP = -(-N // (NS * C)) * (NS * C)
    layer_sc = _layer_sc(NP, D, EP, NR)

    bn = 400
    order = jnp.stack([jnp.arange(D // 2), jnp.arange(D // 2) + D // 2],
                      axis=1).reshape(-1)
    x = node_emb
    for l in range(L):
        wcat = jnp.transpose(W[l], (1, 0, 2))[:, :, order].reshape(D, R * D)
        y = _matmul(x, wcat, bn)
        y_i32 = lax.bitcast_convert_type(y.reshape(NR, D // 2, 2), jnp.int32)
        part = layer_sc(y_i32, g2d, dstp, s2d)
        x = _combine(part, x, W_root[l], b[l], relu=(l < L - 1), bn=bn)
    return x


# speedup vs baseline: 30.6717x; 30.6717x over previous
import functools

import jax
import jax.numpy as jnp
from jax import lax
from jax.experimental import pallas as pl
from jax.experimental.pallas import tpu as pltpu
from jax.experimental.pallas import tpu_sc as plsc

NC = 2
NS = 16
NW = NC * NS
C = 128
G = 8


def _pre_sc(E, EP, R, NSEGP):
    ROWS = EP // C
    RPT_A = ROWS // NS
    RPT_B = ROWS // NW
    GA = 16
    GB = 14 if (ROWS // NW) % 14 == 0 else G
    ZB = 1024
    PS = NSEGP // NS

    def body(srcp, dstp, etp, vald, s_out, g_out,
             counts_sh, srcb, dbuf, ebuf, vbuf, segb, gidxb, cntb, soutb,
             zb, sem):
        cid = lax.axis_index("c")
        sid = lax.axis_index("s")
        w = sid * NC + cid

        for k in range(ZB // 16):
            zb[pl.ds(k * 16, 16)] = jnp.zeros((16,), jnp.float32)
        def body_z(z, c):
            pltpu.sync_copy(zb, counts_sh.at[pl.ds(sid * PS + z * ZB, ZB)])
            return c
        lax.fori_loop(0, PS // ZB, body_z, 0)
        plsc.subcore_barrier()

        def body_a(g, c):
            row0 = sid * RPT_A + g * GA
            pltpu.sync_copy(dstp.at[pl.ds(row0, GA)], dbuf)
            pltpu.sync_copy(etp.at[pl.ds(row0, GA)], ebuf)
            pltpu.sync_copy(vald.at[pl.ds(row0, GA)], vbuf)
            for j in range(GA):
                for k in range(C // 16):
                    sl = pl.ds(k * 16, 16)
                    segb[j, sl] = dbuf[j, sl] * R + ebuf[j, sl]
            cps = [pltpu.async_copy(vbuf.at[j], counts_sh.at[segb.at[j]],
                                    sem, add=True) for j in range(GA)]
            for cp in cps:
                cp.wait()
            return c
        lax.fori_loop(0, RPT_A // GA, body_a, 0)
        plsc.subcore_barrier()

        def body_b(g, c):
            row0 = w * RPT_B + g * GB
            pltpu.sync_copy(srcp.at[pl.ds(row0, GB)], srcb.at[pl.ds(0, GB)])
            pltpu.sync_copy(dstp.at[pl.ds(row0, GB)], dbuf.at[pl.ds(0, GB)])
            pltpu.sync_copy(etp.at[pl.ds(row0, GB)], ebuf.at[pl.ds(0, GB)])
            pltpu.sync_copy(vald.at[pl.ds(row0, GB)], vbuf.at[pl.ds(0, GB)])
            for j in range(GB):
                for k in range(C // 16):
                    sl = pl.ds(k * 16, 16)
                    segb[j, sl] = dbuf[j, sl] * R + ebuf[j, sl]
                    gidxb[j, sl] = srcb[j, sl] * R + ebuf[j, sl]
            cps = [pltpu.async_copy(counts_sh.at[segb.at[j]], cntb.at[j], sem)
                   for j in range(GB)]
            for cp in cps:
                cp.wait()
            for j in range(GB):
                for k in range(C // 16):
                    sl = pl.ds(k * 16, 16)
                    soutb[j, sl] = vbuf[j, sl] / jnp.maximum(cntb[j, sl], 1.0)
            pltpu.sync_copy(soutb.at[pl.ds(0, GB)], s_out.at[pl.ds(row0, GB)])
            pltpu.sync_copy(gidxb.at[pl.ds(0, GB)], g_out.at[pl.ds(row0, GB)])
            return c
        lax.fori_loop(0, RPT_B // GB, body_b, 0)

    mesh = plsc.VectorSubcoreMesh(core_axis_name="c", subcore_axis_name="s")
    return pl.kernel(
        body,
        out_type=(jax.ShapeDtypeStruct((ROWS, C), jnp.float32),
                  jax.ShapeDtypeStruct((ROWS, C), jnp.int32)),
        mesh=mesh,
        compiler_params=pltpu.CompilerParams(use_tc_tiling_on_sc=False),
        scratch_types=[
            pltpu.VMEM_SHARED((NSEGP,), jnp.float32),
            pltpu.VMEM((GA, C), jnp.int32),
            pltpu.VMEM((GA, C), jnp.int32),
            pltpu.VMEM((GA, C), jnp.int32),
            pltpu.VMEM((GA, C), jnp.float32),
            pltpu.VMEM((GA, C), jnp.int32),
            pltpu.VMEM((GA, C), jnp.int32),
            pltpu.VMEM((GA, C), jnp.float32),
            pltpu.VMEM((GA, C), jnp.float32),
            pltpu.VMEM((ZB,), jnp.float32),
            pltpu.SemaphoreType.DMA,
        ],
    )


def _layer_sc(NP, D, EP, NR):
    ROWS = EP // C
    RPT = ROWS // NW
    NPT = NP // NS
    ZR = C

    NG = RPT // G

    def body(y, g2d, dstp, s2d, out,
             acc_sh, gbufA, dbufA, sbufA, gbufB, dbufB, sbufB,
             gr0, gr1, fr0, fr1, sem0, sem1, sem_s, esemA, esemB):
        cid = lax.axis_index("c")
        sid = lax.axis_index("s")
        w = sid * NC + cid

        def body_z0(e, c):
            fr0[e, pl.ds(0, 16)] = jnp.zeros((16,), jnp.float32)
            fr0[e, pl.ds(16, 16)] = jnp.zeros((16,), jnp.float32)
            return c
        lax.fori_loop(0, ZR, body_z0, 0)
        def body_z(z, c):
            pltpu.sync_copy(fr0.at[pl.ds(0, ZR)],
                            acc_sh.at[pl.ds(sid * NPT + z * ZR, ZR)])
            return c
        lax.fori_loop(0, NPT // ZR, body_z, 0)
        plsc.subcore_barrier()

        gbufs = (gr0, gr1)
        fbufs = (fr0, fr1)
        sems = (sem0, sem1)

        def scale(grows, frows, sbuf, j):
            def scale_body(k, c2):
                sv = sbuf[j, pl.ds(k * 16, 16)]
                for i in range(16):
                    e = k * 16 + i
                    si = sv[i]
                    frows[e, pl.ds(0, 16)] = grows[e, pl.ds(0, 16)] * si
                    frows[e, pl.ds(16, 16)] = grows[e, pl.ds(16, 16)] * si
                return c2
            lax.fori_loop(0, C // 16, scale_body, 0)

        setA = (gbufA, dbufA, sbufA)
        setB = (gbufB, dbufB, sbufB)

        def prefetch(eset, esem, g):
            row0 = w * RPT + g * G
            pltpu.async_copy(g2d.at[pl.ds(row0, G)], eset[0], esem)
            pltpu.async_copy(dstp.at[pl.ds(row0, G)], eset[1], esem)
            pltpu.async_copy(s2d.at[pl.ds(row0, G)], eset[2], esem)

        def drain_e(eset, esem):
            pltpu.make_async_copy(g2d.at[pl.ds(0, G)], eset[0], esem).wait()
            pltpu.make_async_copy(dstp.at[pl.ds(0, G)], eset[1], esem).wait()
            pltpu.make_async_copy(s2d.at[pl.ds(0, G)], eset[2], esem).wait()

        def proc(eset):
            gbuf, dbuf, sbuf = eset
            cp = pltpu.async_copy(y.at[gbuf.at[0]], gbufs[0], sems[0])
            scs = []
            for j in range(G):
                if j + 1 < G:
                    cp_n = pltpu.async_copy(y.at[gbuf.at[j + 1]],
                                            gbufs[(j + 1) % 2],
                                            sems[(j + 1) % 2])
                cp.wait()
                if j >= 2:
                    scs[j - 2].wait()
                scale(gbufs[j % 2], fbufs[j % 2], sbuf, j)
                scs.append(pltpu.async_copy(fbufs[j % 2],
                                            acc_sh.at[dbuf.at[j]],
                                            sem_s, add=True))
                if j + 1 < G:
                    cp = cp_n
            scs[G - 2].wait()
            scs[G - 1].wait()

        prefetch(setA, esemA, 0)
        drain_e(setA, esemA)
        prefetch(setB, esemB, 1)

        def pair_body(k, c):
            proc(setA)
            drain_e(setB, esemB)
            prefetch(setA, esemA, 2 * k + 2)
            proc(setB)
            drain_e(setA, esemA)
            prefetch(setB, esemB, jnp.minimum(2 * k + 3, NG - 1))
            return c
        lax.fori_loop(0, (NG - 1) // 2, pair_body, 0)
        proc(setA)
        drain_e(setB, esemB)
        plsc.subcore_barrier()
        def body_o(z, c):
            off = sid * NPT + z * ZR
            pltpu.sync_copy(acc_sh.at[pl.ds(off, ZR)], fr0.at[pl.ds(0, ZR)])
            pltpu.sync_copy(fr0.at[pl.ds(0, ZR)], out.at[cid, pl.ds(off, ZR)])
            return c
        lax.fori_loop(0, NPT // ZR, body_o, 0)

    mesh = plsc.VectorSubcoreMesh(core_axis_name="c", subcore_axis_name="s")
    return pl.kernel(
        body,
        out_type=jax.ShapeDtypeStruct((NC, NP, D), jnp.float32),
        mesh=mesh,
        compiler_params=pltpu.CompilerParams(use_tc_tiling_on_sc=False),
        scratch_types=[
            pltpu.VMEM_SHARED((NP, D), jnp.float32),
            pltpu.VMEM((G, C), jnp.int32),
            pltpu.VMEM((G, C), jnp.int32),
            pltpu.VMEM((G, C), jnp.float32),
            pltpu.VMEM((G, C), jnp.int32),
            pltpu.VMEM((G, C), jnp.int32),
            pltpu.VMEM((G, C), jnp.float32),
            pltpu.VMEM((C, D), jnp.float32),
            pltpu.VMEM((C, D), jnp.float32),
            pltpu.VMEM((C, D), jnp.float32),
            pltpu.VMEM((C, D), jnp.float32),
            pltpu.SemaphoreType.DMA,
            pltpu.SemaphoreType.DMA,
            pltpu.SemaphoreType.DMA,
            pltpu.SemaphoreType.DMA,
            pltpu.SemaphoreType.DMA,
        ],
    )


def _mm_body(x_ref, w_ref, o_ref):
    o_ref[...] = lax.dot_general(
        x_ref[...], w_ref[...], (((1,), (0,)), ((), ())),
        precision=lax.Precision.HIGHEST, preferred_element_type=jnp.float32)


def _matmul(x, w, bn):
    n, d = x.shape
    _, m = w.shape
    return pl.pallas_call(
        _mm_body,
        grid=(n // bn,),
        in_specs=[pl.BlockSpec((bn, d), lambda i: (i, 0)),
                  pl.BlockSpec((d, m), lambda i: (0, 0))],
        out_specs=pl.BlockSpec((bn, m), lambda i: (i, 0)),
        out_shape=jax.ShapeDtypeStruct((n, m), jnp.float32),
    )(x, w)


def _comb_body(a_ref, x_ref, wr_ref, b_ref, o_ref, *, relu):
    v = (a_ref[0] + a_ref[1] + b_ref[...]
         + lax.dot_general(x_ref[...], wr_ref[...], (((1,), (0,)), ((), ())),
                           precision=lax.Precision.HIGHEST,
                           preferred_element_type=jnp.float32))
    o_ref[...] = jnp.maximum(v, 0.0) if relu else v


def _combine(part, x, wr, bvec, relu, bn):
    n, d = x.shape
    return pl.pallas_call(
        functools.partial(_comb_body, relu=relu),
        grid=(n // bn,),
        in_specs=[pl.BlockSpec((2, bn, d), lambda i: (0, i, 0)),
                  pl.BlockSpec((bn, d), lambda i: (i, 0)),
                  pl.BlockSpec((d, d), lambda i: (0, 0)),
                  pl.BlockSpec((1, d), lambda i: (0, 0))],
        out_specs=pl.BlockSpec((bn, d), lambda i: (i, 0)),
        out_shape=jax.ShapeDtypeStruct((n, d), jnp.float32),
    )(part, x, wr, bvec.reshape(1, d))


def kernel(edge_index, edge_type, node_emb, W, W_root, b):
    N, D = node_emb.shape
    L, R = W.shape[0], W.shape[1]
    E = edge_type.shape[0]
    NR = N * R

    EP = -(-E // (NW * G * C)) * (NW * G * C)
    ROWS = EP // C
    NSEGP = -(-NR // (NS * 1024)) * (NS * 1024)
    pad = EP - E

    src = edge_index[0]
    dst = edge_index[1]
    zi = jnp.zeros((pad,), jnp.int32)
    srcp = jnp.concatenate([src, zi]).reshape(ROWS, C)
    dstp = jnp.concatenate([dst, zi]).reshape(ROWS, C)
    etp = jnp.concatenate([edge_type, zi]).reshape(ROWS, C)
    vald = jnp.concatenate([jnp.ones((E,), jnp.float32),
                            jnp.zeros((pad,), jnp.float32)]).reshape(ROWS, C)

    s2d, g2d = _pre_sc(E, EP, R, NSEGP)(srcp, dstp, etp, vald)
    NP = -(-N // (NS * C)) * (NS * C)
    layer_sc = _layer_sc(NP, D, EP, NR)

    bn = 400
    x = node_emb
    for l in range(L):
        wcat = jnp.transpose(W[l], (1, 0, 2)).reshape(D, R * D)
        y = _matmul(x, wcat, bn)
        part = layer_sc(y.reshape(NR, D), g2d, dstp, s2d)
        x = _combine(part, x, W_root[l], b[l], relu=(l < L - 1), bn=bn)
    return x
